# baseline (speedup 1.0000x reference)
import functools

import jax
import jax.numpy as jnp
from jax import lax
from jax.experimental import pallas as pl
from jax.experimental.pallas import tpu as pltpu
from jax.experimental.pallas import tpu_sc as plsc

NN = 10000
DIN = 128
DHID = 128
NCLS = 40
NE = 320000
ETOT = NE + NN
EPS = 1e-5

C = 128
NCHUNK = 2624
EPAD = NCHUNK * C
NACC = 10144
STRIPE = NACC // 16
BLK = 64
D2P = 48

_mesh = plsc.VectorSubcoreMesh(core_axis_name="c", subcore_axis_name="s")
_f32 = jnp.float32
_sc_params = pltpu.CompilerParams(use_tc_tiling_on_sc=False)

KDEG = NCHUNK // 32


def _stripe_pieces():
    off = 0
    while off < STRIPE:
        n = min(C, STRIPE - off)
        yield off, n
        off += n


@functools.partial(
    pl.kernel,
    out_type=jax.ShapeDtypeStruct((2, NACC, 16), _f32),
    mesh=_mesh,
    scratch_types=[
        pltpu.VMEM((C, 16), _f32),
        pltpu.VMEM((STRIPE, 16), _f32),
        pltpu.VMEM((KDEG, C), jnp.int32),
        pltpu.VMEM_SHARED((NACC, 16), _f32),
    ],
    compiler_params=_sc_params,
)
def _deg_kernel(dst_hbm, out_hbm, ones_v, zbuf_v, didx_v, acc):
    cid = lax.axis_index("c")
    sid = lax.axis_index("s")
    wid = sid * 2 + cid

    @pl.loop(0, C)
    def _(i):
        ones_v[i, :] = jnp.ones((16,), _f32)

    @pl.loop(0, STRIPE)
    def _(i):
        zbuf_v[i, :] = jnp.zeros((16,), _f32)

    pltpu.sync_copy(dst_hbm.at[pl.ds(wid * KDEG, KDEG)], didx_v)
    pltpu.sync_copy(zbuf_v, acc.at[pl.ds(sid * STRIPE, STRIPE)])
    plsc.subcore_barrier()

    @pl.loop(0, KDEG)
    def _(j):
        pltpu.sync_copy(ones_v, acc.at[didx_v.at[j]], add=True)

    plsc.subcore_barrier()
    pltpu.sync_copy(acc.at[pl.ds(sid * STRIPE, STRIPE)],
                    out_hbm.at[cid, pl.ds(sid * STRIPE, STRIPE)])


def _make_pass_kernel(d, k0, k1):
    assert 16 * (k0 + k1) == NCHUNK
    nparts = 1 if k1 == 0 else 2
    out_shape = (NACC, d) if nparts == 1 else (nparts, NACC, d)

    @functools.partial(
        pl.kernel,
        out_type=jax.ShapeDtypeStruct(out_shape, _f32),
        mesh=_mesh,
        scratch_types=[
            pltpu.VMEM((BLK, C), jnp.int32),
            pltpu.VMEM((C,), jnp.int32),
            pltpu.VMEM((C,), jnp.int32),
            pltpu.VMEM((C, d), _f32),
            pltpu.VMEM((C, d), _f32),
            pltpu.VMEM_SHARED((NACC, d), _f32),
            pltpu.SemaphoreType.DMA,
            pltpu.SemaphoreType.DMA,
        ],
        compiler_params=_sc_params,
    )
    def _pass(src_hbm, dst_hbm, table_hbm, out_hbm,
              sidx_v, didx0_v, didx1_v, rows0_v, rows1_v, acc, sem0, sem1):
        cid = lax.axis_index("c")
        sid = lax.axis_index("s")

        def prep():
            @pl.loop(0, C)
            def _(i):
                @pl.loop(0, d, step=16)
                def _(j):
                    rows0_v[i, pl.ds(j, 16)] = jnp.zeros((16,), _f32)

            for off, n in _stripe_pieces():
                pltpu.sync_copy(
                    rows0_v.at[pl.ds(0, n)],
                    acc.at[pl.ds(sid * STRIPE + off, n)])
            plsc.subcore_barrier()

        def edge_block(kk, base):
            pltpu.sync_copy(src_hbm.at[pl.ds(base, kk)],
                            sidx_v.at[pl.ds(0, kk)])
            pltpu.async_copy(table_hbm.at[sidx_v.at[0]], rows0_v, sem0)

            @pl.loop(0, kk - 2, step=2)
            def _(j):
                pltpu.async_copy(
                    table_hbm.at[sidx_v.at[j + 1]], rows1_v, sem1)
                pltpu.make_async_copy(
                    table_hbm.at[sidx_v.at[j]], rows0_v, sem0).wait()
                pltpu.sync_copy(dst_hbm.at[base + j], didx0_v)
                pltpu.sync_copy(rows0_v, acc.at[didx0_v], add=True)
                pltpu.async_copy(
                    table_hbm.at[sidx_v.at[j + 2]], rows0_v, sem0)
                pltpu.make_async_copy(
                    table_hbm.at[sidx_v.at[j + 1]], rows1_v, sem1).wait()
                pltpu.sync_copy(dst_hbm.at[base + j + 1], didx1_v)
                pltpu.sync_copy(rows1_v, acc.at[didx1_v], add=True)

            pltpu.async_copy(table_hbm.at[sidx_v.at[kk - 1]], rows1_v, sem1)
            pltpu.make_async_copy(
                table_hbm.at[sidx_v.at[kk - 2]], rows0_v, sem0).wait()
            pltpu.sync_copy(dst_hbm.at[base + kk - 2], didx0_v)
            pltpu.sync_copy(rows0_v, acc.at[didx0_v], add=True)
            pltpu.make_async_copy(
                table_hbm.at[sidx_v.at[kk - 1]], rows1_v, sem1).wait()
            pltpu.sync_copy(dst_hbm.at[base + kk - 1], didx1_v)
            pltpu.sync_copy(rows1_v, acc.at[didx1_v], add=True)

        def edge_loop(kk, base):
            for off in range(0, kk, BLK):
                edge_block(min(BLK, kk - off), base + off)

        if nparts == 1:
            @pl.when(cid == 0)
            def _():
                prep()
                edge_loop(k0, sid * k0)
                plsc.subcore_barrier()
                for off, n in _stripe_pieces():
                    pltpu.sync_copy(
                        acc.at[pl.ds(sid * STRIPE + off, n)],
                        out_hbm.at[pl.ds(sid * STRIPE + off, n)])
        else:
            prep()

            @pl.when(cid == 0)
            def _():
                edge_loop(k0, sid * k0)

            @pl.when(cid == 1)
            def _():
                edge_loop(k1, 16 * k0 + sid * k1)

            plsc.subcore_barrier()
            for off, n in _stripe_pieces():
                pltpu.sync_copy(
                    acc.at[pl.ds(sid * STRIPE + off, n)],
                    out_hbm.at[cid, pl.ds(sid * STRIPE + off, n)])

    return _pass


_pass128 = _make_pass_kernel(DHID, 164, 0)
_pass48 = _make_pass_kernel(D2P, 114, 50)


_R = 400


def _tc1_body(x_ref, w1_ref, g0_ref, b0_ref, degp_ref, o_ref):
    deg = degp_ref[0, :, 0] + degp_ref[1, :, 0]
    dinv = lax.rsqrt(jnp.maximum(deg, 1.0))
    s0 = g0_ref[...] * lax.rsqrt(jnp.float32(1.0 + EPS))
    h = x_ref[...] * s0[None, :] + b0_ref[...][None, :]
    u = jnp.dot(h, w1_ref[...], preferred_element_type=_f32)
    o_ref[...] = u * dinv[:, None]


def _tc2_body(p_ref, degp_ref, w2_ref, b1_ref, g1_ref, bb1_ref, o_ref):
    deg = degp_ref[0, :, 0] + degp_ref[1, :, 0]
    dinv = lax.rsqrt(jnp.maximum(deg, 1.0))
    agg = p_ref[...] * dinv[:, None] + b1_ref[...][None, :]
    s1 = g1_ref[...] * lax.rsqrt(jnp.float32(1.0 + EPS))
    h = jnp.maximum(agg * s1[None, :] + bb1_ref[...][None, :], 0.0)
    u = jnp.dot(h, w2_ref[...], preferred_element_type=_f32)
    u = u * dinv[:, None]
    o_ref[...] = jnp.concatenate(
        [u, jnp.zeros((_R, D2P - NCLS), _f32)], axis=1)


def _tc3_body(q_ref, degp_ref, b2_ref, o_ref):
    deg = degp_ref[0, :, 0] + degp_ref[1, :, 0]
    dinv = lax.rsqrt(jnp.maximum(deg, 1.0))
    o_ref[...] = ((q_ref[0] + q_ref[1])[:, :NCLS] * dinv[:, None]
                  + b2_ref[...][None, :])


def _row_spec(d):
    return pl.BlockSpec((_R, d), lambda i: (i, 0))


_degp_spec = pl.BlockSpec((2, _R, 16), lambda i: (0, i, 0))


def _full_spec(shape):
    nd = len(shape)
    return pl.BlockSpec(shape, lambda i: (0,) * nd)


def _tc1(x, w1, g0, b0, degp):
    return pl.pallas_call(
        _tc1_body,
        grid=(NN // _R,),
        in_specs=[_row_spec(DIN), _full_spec((DIN, DHID)),
                  _full_spec((DIN,)), _full_spec((DIN,)), _degp_spec],
        out_specs=_row_spec(DHID),
        out_shape=jax.ShapeDtypeStruct((NN, DHID), _f32),
    )(x, w1, g0, b0, degp)


def _tc2(p, degp, w2, b1, g1, bb1):
    return pl.pallas_call(
        _tc2_body,
        grid=(NN // _R,),
        in_specs=[_row_spec(DHID),
                  _degp_spec, _full_spec((DHID, NCLS)),
                  _full_spec((DHID,)), _full_spec((DHID,)),
                  _full_spec((DHID,))],
        out_specs=_row_spec(D2P),
        out_shape=jax.ShapeDtypeStruct((NN, D2P), _f32),
    )(p, degp, w2, b1, g1, bb1)


def _tc3(q, degp, b2):
    return pl.pallas_call(
        _tc3_body,
        grid=(NN // _R,),
        in_specs=[pl.BlockSpec((2, _R, D2P), lambda i: (0, i, 0)),
                  _degp_spec, _full_spec((NCLS,))],
        out_specs=_row_spec(NCLS),
        out_shape=jax.ShapeDtypeStruct((NN, NCLS), _f32),
    )(q, degp, b2)


def kernel(x, edge_index, bn0_gamma, bn0_beta, W1, b1, bn1_gamma, bn1_beta,
           W2, b2):
    sl = jnp.arange(NN, dtype=jnp.int32)
    npad = EPAD - ETOT
    src = jnp.concatenate(
        [edge_index[0].astype(jnp.int32), sl,
         jnp.zeros((npad,), jnp.int32)]).reshape(NCHUNK, C)
    pad_dst = NN + (jnp.arange(npad, dtype=jnp.int32) % (NACC - NN))
    dst = jnp.concatenate(
        [edge_index[1].astype(jnp.int32), sl, pad_dst]).reshape(NCHUNK, C)

    degp = _deg_kernel(dst)
    hw1 = _tc1(x, W1, bn0_gamma, bn0_beta, degp)
    p = _pass128(src, dst, hw1)
    hw2 = _tc2(p, degp, W2, b1, bn1_gamma, bn1_beta)
    q = _pass48(src, dst, hw2)
    return _tc3(q, degp, b2)


# speedup vs baseline: 1.3195x; 1.3195x over previous
import functools

import jax
import jax.numpy as jnp
from jax import lax
from jax.experimental import pallas as pl
from jax.experimental.pallas import tpu as pltpu
from jax.experimental.pallas import tpu_sc as plsc

NN = 10000
DIN = 128
DHID = 128
NCLS = 40
NE = 320000
ETOT = NE + NN
EPS = 1e-5

C = 128
NCHUNK = 2624
EPAD = NCHUNK * C
NACC = 10240
STRIPE = NACC // 16
BLK = 82
D2P = 48

_mesh = plsc.VectorSubcoreMesh(core_axis_name="c", subcore_axis_name="s")
_f32 = jnp.float32
_sc_params = pltpu.CompilerParams(use_tc_tiling_on_sc=False)

KDEG = NCHUNK // 32


def _stripe_pieces():
    off = 0
    while off < STRIPE:
        n = min(C, STRIPE - off)
        yield off, n
        off += n


@functools.partial(
    pl.kernel,
    out_type=jax.ShapeDtypeStruct((2, NACC, 16), _f32),
    mesh=_mesh,
    scratch_types=[
        pltpu.VMEM((C, 16), _f32),
        pltpu.VMEM((STRIPE, 16), _f32),
        pltpu.VMEM((KDEG, C), jnp.int32),
        pltpu.VMEM_SHARED((NACC, 16), _f32),
    ],
    compiler_params=_sc_params,
)
def _deg_kernel(dst_hbm, out_hbm, ones_v, zbuf_v, didx_v, acc):
    cid = lax.axis_index("c")
    sid = lax.axis_index("s")
    wid = sid * 2 + cid

    @pl.loop(0, C)
    def _(i):
        ones_v[i, :] = jnp.ones((16,), _f32)

    @pl.loop(0, STRIPE)
    def _(i):
        zbuf_v[i, :] = jnp.zeros((16,), _f32)

    pltpu.sync_copy(dst_hbm.at[pl.ds(wid * KDEG, KDEG)], didx_v)
    pltpu.sync_copy(zbuf_v, acc.at[pl.ds(sid * STRIPE, STRIPE)])
    plsc.subcore_barrier()

    @pl.loop(0, KDEG)
    def _(j):
        pltpu.sync_copy(ones_v, acc.at[didx_v.at[j]], add=True)

    plsc.subcore_barrier()
    pltpu.sync_copy(acc.at[pl.ds(sid * STRIPE, STRIPE)],
                    out_hbm.at[cid, pl.ds(sid * STRIPE, STRIPE)])


def _make_pass_kernel(d, k0, k1):
    assert 16 * (k0 + k1) == NCHUNK
    nparts = 1 if k1 == 0 else 2
    out_shape = (NACC, d) if nparts == 1 else (nparts, NACC, d)

    @functools.partial(
        pl.kernel,
        out_type=jax.ShapeDtypeStruct(out_shape, _f32),
        mesh=_mesh,
        scratch_types=[
            pltpu.VMEM((BLK, C), jnp.int32),
            pltpu.VMEM((C,), jnp.int32),
            pltpu.VMEM((C,), jnp.int32),
            pltpu.VMEM((C, d), _f32),
            pltpu.VMEM((C, d), _f32),
            pltpu.VMEM_SHARED((NACC, d), _f32),
            pltpu.SemaphoreType.DMA,
            pltpu.SemaphoreType.DMA,
        ],
        compiler_params=_sc_params,
    )
    def _pass(src_hbm, dst_hbm, table_hbm, out_hbm,
              sidx_v, didx0_v, didx1_v, rows0_v, rows1_v, acc, sem0, sem1):
        cid = lax.axis_index("c")
        sid = lax.axis_index("s")

        def prep():
            @pl.loop(0, C)
            def _(i):
                @pl.loop(0, d, step=16)
                def _(j):
                    rows0_v[i, pl.ds(j, 16)] = jnp.zeros((16,), _f32)

            for off, n in _stripe_pieces():
                pltpu.sync_copy(
                    rows0_v.at[pl.ds(0, n)],
                    acc.at[pl.ds(sid * STRIPE + off, n)])
            plsc.subcore_barrier()

        def edge_block(kk, base):
            pltpu.sync_copy(src_hbm.at[pl.ds(base, kk)],
                            sidx_v.at[pl.ds(0, kk)])
            pltpu.async_copy(table_hbm.at[sidx_v.at[0]], rows0_v, sem0)

            @pl.loop(0, kk - 2, step=2)
            def _(j):
                pltpu.async_copy(
                    table_hbm.at[sidx_v.at[j + 1]], rows1_v, sem1)
                pltpu.make_async_copy(
                    table_hbm.at[sidx_v.at[j]], rows0_v, sem0).wait()
                pltpu.sync_copy(dst_hbm.at[base + j], didx0_v)
                pltpu.sync_copy(rows0_v, acc.at[didx0_v], add=True)
                pltpu.async_copy(
                    table_hbm.at[sidx_v.at[j + 2]], rows0_v, sem0)
                pltpu.make_async_copy(
                    table_hbm.at[sidx_v.at[j + 1]], rows1_v, sem1).wait()
                pltpu.sync_copy(dst_hbm.at[base + j + 1], didx1_v)
                pltpu.sync_copy(rows1_v, acc.at[didx1_v], add=True)

            pltpu.async_copy(table_hbm.at[sidx_v.at[kk - 1]], rows1_v, sem1)
            pltpu.make_async_copy(
                table_hbm.at[sidx_v.at[kk - 2]], rows0_v, sem0).wait()
            pltpu.sync_copy(dst_hbm.at[base + kk - 2], didx0_v)
            pltpu.sync_copy(rows0_v, acc.at[didx0_v], add=True)
            pltpu.make_async_copy(
                table_hbm.at[sidx_v.at[kk - 1]], rows1_v, sem1).wait()
            pltpu.sync_copy(dst_hbm.at[base + kk - 1], didx1_v)
            pltpu.sync_copy(rows1_v, acc.at[didx1_v], add=True)

        def edge_loop(kk, base):
            for off in range(0, kk, BLK):
                edge_block(min(BLK, kk - off), base + off)

        if nparts == 1:
            @pl.when(cid == 0)
            def _():
                prep()
                edge_loop(k0, sid * k0)
                plsc.subcore_barrier()
                for off, n in _stripe_pieces():
                    pltpu.sync_copy(
                        acc.at[pl.ds(sid * STRIPE + off, n)],
                        out_hbm.at[pl.ds(sid * STRIPE + off, n)])
        else:
            prep()

            @pl.when(cid == 0)
            def _():
                edge_loop(k0, sid * k0)

            @pl.when(cid == 1)
            def _():
                edge_loop(k1, 16 * k0 + sid * k1)

            plsc.subcore_barrier()
            for off, n in _stripe_pieces():
                pltpu.sync_copy(
                    acc.at[pl.ds(sid * STRIPE + off, n)],
                    out_hbm.at[cid, pl.ds(sid * STRIPE + off, n)])

    return _pass


_pass128 = _make_pass_kernel(DHID, 146, 18)
_pass48 = _make_pass_kernel(D2P, 114, 50)


_R = 400


def _tc1_body(x_ref, w1_ref, g0_ref, b0_ref, degp_ref, o_ref):
    deg = degp_ref[0, :, 0] + degp_ref[1, :, 0]
    dinv = lax.rsqrt(jnp.maximum(deg, 1.0))
    s0 = g0_ref[...] * lax.rsqrt(jnp.float32(1.0 + EPS))
    h = x_ref[...] * s0[None, :] + b0_ref[...][None, :]
    u = jnp.dot(h, w1_ref[...], preferred_element_type=_f32)
    o_ref[...] = u * dinv[:, None]


def _tc2_body(p_ref, degp_ref, w2_ref, b1_ref, g1_ref, bb1_ref, o_ref):
    deg = degp_ref[0, :, 0] + degp_ref[1, :, 0]
    dinv = lax.rsqrt(jnp.maximum(deg, 1.0))
    agg = (p_ref[0] + p_ref[1]) * dinv[:, None] + b1_ref[...][None, :]
    s1 = g1_ref[...] * lax.rsqrt(jnp.float32(1.0 + EPS))
    h = jnp.maximum(agg * s1[None, :] + bb1_ref[...][None, :], 0.0)
    u = jnp.dot(h, w2_ref[...], preferred_element_type=_f32)
    u = u * dinv[:, None]
    o_ref[...] = jnp.concatenate(
        [u, jnp.zeros((_R, D2P - NCLS), _f32)], axis=1)


def _tc3_body(q_ref, degp_ref, b2_ref, o_ref):
    deg = degp_ref[0, :, 0] + degp_ref[1, :, 0]
    dinv = lax.rsqrt(jnp.maximum(deg, 1.0))
    o_ref[...] = ((q_ref[0] + q_ref[1])[:, :NCLS] * dinv[:, None]
                  + b2_ref[...][None, :])


def _row_spec(d):
    return pl.BlockSpec((_R, d), lambda i: (i, 0))


_degp_spec = pl.BlockSpec((2, _R, 16), lambda i: (0, i, 0))


def _full_spec(shape):
    nd = len(shape)
    return pl.BlockSpec(shape, lambda i: (0,) * nd)


def _tc1(x, w1, g0, b0, degp):
    return pl.pallas_call(
        _tc1_body,
        grid=(NN // _R,),
        in_specs=[_row_spec(DIN), _full_spec((DIN, DHID)),
                  _full_spec((DIN,)), _full_spec((DIN,)), _degp_spec],
        out_specs=_row_spec(DHID),
        out_shape=jax.ShapeDtypeStruct((NN, DHID), _f32),
    )(x, w1, g0, b0, degp)


def _tc2(p, degp, w2, b1, g1, bb1):
    return pl.pallas_call(
        _tc2_body,
        grid=(NN // _R,),
        in_specs=[pl.BlockSpec((2, _R, DHID), lambda i: (0, i, 0)),
                  _degp_spec, _full_spec((DHID, NCLS)),
                  _full_spec((DHID,)), _full_spec((DHID,)),
                  _full_spec((DHID,))],
        out_specs=_row_spec(D2P),
        out_shape=jax.ShapeDtypeStruct((NN, D2P), _f32),
    )(p, degp, w2, b1, g1, bb1)


def _tc3(q, degp, b2):
    return pl.pallas_call(
        _tc3_body,
        grid=(NN // _R,),
        in_specs=[pl.BlockSpec((2, _R, D2P), lambda i: (0, i, 0)),
                  _degp_spec, _full_spec((NCLS,))],
        out_specs=_row_spec(NCLS),
        out_shape=jax.ShapeDtypeStruct((NN, NCLS), _f32),
    )(q, degp, b2)


def kernel(x, edge_index, bn0_gamma, bn0_beta, W1, b1, bn1_gamma, bn1_beta,
           W2, b2):
    sl = jnp.arange(NN, dtype=jnp.int32)
    npad = EPAD - ETOT
    src = jnp.concatenate(
        [edge_index[0].astype(jnp.int32), sl,
         jnp.zeros((npad,), jnp.int32)]).reshape(NCHUNK, C)
    pad_dst = NN + (jnp.arange(npad, dtype=jnp.int32) % (NACC - NN))
    dst = jnp.concatenate(
        [edge_index[1].astype(jnp.int32), sl, pad_dst]).reshape(NCHUNK, C)

    degp = _deg_kernel(dst)
    hw1 = _tc1(x, W1, bn0_gamma, bn0_beta, degp)
    p = _pass128(src, dst, hw1)
    hw2 = _tc2(p, degp, W2, b1, bn1_gamma, bn1_beta)
    q = _pass48(src, dst, hw2)
    return _tc3(q, degp, b2)
